# baseline (device time: 192557 ns/iter reference)
import jax
import jax.numpy as jnp
from jax import lax
from jax.experimental import pallas as pl
from jax.experimental.pallas import tpu as pltpu

N_DEV = 8


def kernel(x, w_mat):
    m, k_loc = x.shape
    _, n = w_mat.shape
    mc = m // N_DEV
    n_hops = N_DEV - 1

    def body(x_ref, w_ref, out_ref, w_bf_ref, send_ref, recv_ref,
             send_sems, recv_sems):
        my = lax.axis_index("i")
        left = jnp.mod(my - 1 + N_DEV, N_DEV)
        right = jnp.mod(my + 1, N_DEV)

        barrier_sem = pltpu.get_barrier_semaphore()
        for nbr in (left, right):
            pl.semaphore_signal(
                barrier_sem, inc=1,
                device_id=(nbr,), device_id_type=pl.DeviceIdType.MESH,
            )
        pl.semaphore_wait(barrier_sem, 2)

        w_bf_ref[:, :] = w_ref[:, :].astype(jnp.bfloat16)

        def partial(c):
            xc = x_ref[pl.ds(c * mc, mc), :].astype(jnp.bfloat16)
            return jnp.dot(xc, w_bf_ref[:, :],
                           preferred_element_type=jnp.float32)

        for t in range(n_hops):
            c = jnp.mod(my - 1 - t + 2 * N_DEV, N_DEV)
            acc = partial(c)
            if t > 0:
                acc = acc + recv_ref[t - 1].astype(jnp.float32)
            send_ref[t % 2] = acc.astype(jnp.bfloat16)
            rdma = pltpu.make_async_remote_copy(
                src_ref=send_ref.at[t % 2],
                dst_ref=recv_ref.at[t],
                send_sem=send_sems.at[t],
                recv_sem=recv_sems.at[t],
                device_id=(right,),
                device_id_type=pl.DeviceIdType.MESH,
            )
            rdma.start()
            rdma.wait()

        out_ref[:, :] = (recv_ref[n_hops - 1].astype(jnp.float32)
                         + partial(my))

    return pl.pallas_call(
        body,
        out_shape=jax.ShapeDtypeStruct((mc, n), jnp.float32),
        in_specs=[
            pl.BlockSpec(memory_space=pltpu.VMEM),
            pl.BlockSpec(memory_space=pltpu.VMEM),
        ],
        out_specs=pl.BlockSpec(memory_space=pltpu.VMEM),
        scratch_shapes=[
            pltpu.VMEM((k_loc, n), jnp.bfloat16),
            pltpu.VMEM((2, mc, n), jnp.bfloat16),
            pltpu.VMEM((n_hops, mc, n), jnp.bfloat16),
            pltpu.SemaphoreType.DMA((n_hops,)),
            pltpu.SemaphoreType.DMA((n_hops,)),
        ],
        compiler_params=pltpu.CompilerParams(collective_id=0),
    )(x, w_mat)


# device time: 115787 ns/iter; 1.6630x vs baseline; 1.6630x over previous
import jax
import jax.numpy as jnp
from jax import lax
from jax.experimental import pallas as pl
from jax.experimental.pallas import tpu as pltpu

N_DEV = 8


def kernel(x, w_mat):
    m, k_loc = x.shape
    _, n = w_mat.shape
    mc = m // N_DEV
    nh = n // 2
    n_hops = N_DEV - 1

    def body(x_ref, w_ref, out_ref, w_bf_ref,
             sendr_ref, sendl_ref, recvr_ref, recvl_ref,
             send_sems_r, recv_sems_r, send_sems_l, recv_sems_l):
        my = lax.axis_index("i")
        left = jnp.mod(my - 1 + N_DEV, N_DEV)
        right = jnp.mod(my + 1, N_DEV)

        barrier_sem = pltpu.get_barrier_semaphore()
        for nbr in (left, right):
            pl.semaphore_signal(
                barrier_sem, inc=1,
                device_id=(nbr,), device_id_type=pl.DeviceIdType.MESH,
            )
        pl.semaphore_wait(barrier_sem, 2)

        w_bf_ref[:, :] = w_ref[:, :].astype(jnp.bfloat16)

        def partial_r(c):
            xc = x_ref[pl.ds(c * mc, mc), :].astype(jnp.bfloat16)
            return jnp.dot(xc, w_bf_ref[:, :nh],
                           preferred_element_type=jnp.float32)

        def partial_l(c):
            xc = x_ref[pl.ds(c * mc, mc), :].astype(jnp.bfloat16)
            return jnp.dot(xc, w_bf_ref[:, nh:],
                           preferred_element_type=jnp.float32)

        def chunk_r(t):
            return jnp.mod(my - 1 - t + 2 * N_DEV, N_DEV)

        def chunk_l(t):
            return jnp.mod(my + 1 + t, N_DEV)

        p_r = partial_r(chunk_r(0))
        p_l = partial_l(chunk_l(0))
        for t in range(n_hops):
            acc_r = p_r
            acc_l = p_l
            if t > 0:
                acc_r = acc_r + recvr_ref[t - 1].astype(jnp.float32)
                acc_l = acc_l + recvl_ref[t - 1].astype(jnp.float32)
            sendr_ref[t % 2] = acc_r.astype(jnp.bfloat16)
            sendl_ref[t % 2] = acc_l.astype(jnp.bfloat16)
            rdma_r = pltpu.make_async_remote_copy(
                src_ref=sendr_ref.at[t % 2],
                dst_ref=recvr_ref.at[t],
                send_sem=send_sems_r.at[t],
                recv_sem=recv_sems_r.at[t],
                device_id=(right,),
                device_id_type=pl.DeviceIdType.MESH,
            )
            rdma_l = pltpu.make_async_remote_copy(
                src_ref=sendl_ref.at[t % 2],
                dst_ref=recvl_ref.at[t],
                send_sem=send_sems_l.at[t],
                recv_sem=recv_sems_l.at[t],
                device_id=(left,),
                device_id_type=pl.DeviceIdType.MESH,
            )
            rdma_r.start()
            rdma_l.start()
            if t < n_hops - 1:
                p_r = partial_r(chunk_r(t + 1))
                p_l = partial_l(chunk_l(t + 1))
            else:
                p_r = partial_r(my)
                p_l = partial_l(my)
            rdma_r.wait()
            rdma_l.wait()

        out_ref[:, :nh] = recvr_ref[n_hops - 1].astype(jnp.float32) + p_r
        out_ref[:, nh:] = recvl_ref[n_hops - 1].astype(jnp.float32) + p_l

    return pl.pallas_call(
        body,
        out_shape=jax.ShapeDtypeStruct((mc, n), jnp.float32),
        in_specs=[
            pl.BlockSpec(memory_space=pltpu.VMEM),
            pl.BlockSpec(memory_space=pltpu.VMEM),
        ],
        out_specs=pl.BlockSpec(memory_space=pltpu.VMEM),
        scratch_shapes=[
            pltpu.VMEM((k_loc, n), jnp.bfloat16),
            pltpu.VMEM((2, mc, nh), jnp.bfloat16),
            pltpu.VMEM((2, mc, nh), jnp.bfloat16),
            pltpu.VMEM((n_hops, mc, nh), jnp.bfloat16),
            pltpu.VMEM((n_hops, mc, nh), jnp.bfloat16),
            pltpu.SemaphoreType.DMA((n_hops,)),
            pltpu.SemaphoreType.DMA((n_hops,)),
            pltpu.SemaphoreType.DMA((n_hops,)),
            pltpu.SemaphoreType.DMA((n_hops,)),
        ],
        compiler_params=pltpu.CompilerParams(collective_id=0),
    )(x, w_mat)


# device time: 84013 ns/iter; 2.2920x vs baseline; 1.3782x over previous
import jax
import jax.numpy as jnp
from jax import lax
from jax.experimental import pallas as pl
from jax.experimental.pallas import tpu as pltpu

N_DEV = 8

STRIPES = (
    (0, 768, (4, 3, 1)),
    (768, 640, (3, 1, 4)),
    (1408, 640, (1, 4, 3)),
)


def kernel(x, w_mat):
    m, k_loc = x.shape
    _, n = w_mat.shape
    mc = m // N_DEV
    assert n == STRIPES[-1][0] + STRIPES[-1][1]

    def body(x_ref, w_ref, out_ref, x_bf_ref, w_bf_ref, p_ref,
             r1_0, r2_0, r3_0, r1_1, r2_1, r3_1, r1_2, r2_2, r3_2,
             send_sems, recv_sems):
        r1 = (r1_0, r1_1, r1_2)
        r2 = (r2_0, r2_1, r2_2)
        r3 = (r3_0, r3_1, r3_2)

        my = lax.axis_index("i")

        def partner(g):
            return jnp.bitwise_xor(my, g)

        barrier_sem = pltpu.get_barrier_semaphore()
        for g in (1, 3, 4):
            pl.semaphore_signal(
                barrier_sem, inc=1,
                device_id=(partner(g),), device_id_type=pl.DeviceIdType.MESH,
            )
        pl.semaphore_wait(barrier_sem, 3)

        x_bf_ref[:, :] = x_ref[:, :].astype(jnp.bfloat16)
        w_bf_ref[:, :] = w_ref[:, :].astype(jnp.bfloat16)

        def d_list(g2, g3):
            return (0, g3, g2, g2 ^ g3)

        def copy(src, dst, s, j, g):
            return pltpu.make_async_remote_copy(
                src_ref=src, dst_ref=dst,
                send_sem=send_sems.at[s, j], recv_sem=recv_sems.at[s, j],
                device_id=(partner(g),), device_id_type=pl.DeviceIdType.MESH,
            )

        l1 = {}
        for s, (c0, cs, (g1, g2, g3)) in enumerate(STRIPES):
            for c8 in range(N_DEV):
                p_ref[c8 * mc:(c8 + 1) * mc, c0:c0 + cs] = jnp.dot(
                    x_bf_ref[c8 * mc:(c8 + 1) * mc, :],
                    w_bf_ref[:, c0:c0 + cs],
                    preferred_element_type=jnp.float32,
                ).astype(jnp.bfloat16)
            for j, d in enumerate(d_list(g2, g3)):
                c = jnp.bitwise_xor(partner(g1), d)
                rdma = copy(
                    p_ref.at[pl.ds(c * mc, mc), pl.ds(c0, cs)],
                    r1[s].at[j], s, j, g1,
                )
                rdma.start()
                l1[s, j] = rdma

        l2 = {}
        for s, (c0, cs, (g1, g2, g3)) in enumerate(STRIPES):
            for j in range(4):
                l1[s, j].wait()
            for j, d in enumerate(d_list(g2, g3)):
                c = jnp.bitwise_xor(my, d)
                mine = p_ref[pl.ds(c * mc, mc), pl.ds(c0, cs)]
                r1[s][j] = (mine.astype(jnp.float32)
                            + r1[s][j].astype(jnp.float32)).astype(jnp.bfloat16)
            for j in range(2):
                rdma = copy(r1[s].at[2 + j], r2[s].at[j], s, j, g2)
                rdma.start()
                l2[s, j] = rdma

        l3 = {}
        for s, (c0, cs, (g1, g2, g3)) in enumerate(STRIPES):
            for j in range(2):
                l2[s, j].wait()
                r2[s][j] = (r1[s][j].astype(jnp.float32)
                            + r2[s][j].astype(jnp.float32)).astype(jnp.bfloat16)
            rdma = copy(r2[s].at[1], r3[s], s, 0, g3)
            rdma.start()
            l3[s] = rdma

        for s, (c0, cs, _) in enumerate(STRIPES):
            l3[s].wait()
            out_ref[:, c0:c0 + cs] = (r2[s][0].astype(jnp.float32)
                                      + r3[s][:, :].astype(jnp.float32))

    c_sizes = [cs for _, cs, _ in STRIPES]
    return pl.pallas_call(
        body,
        out_shape=jax.ShapeDtypeStruct((mc, n), jnp.float32),
        in_specs=[
            pl.BlockSpec(memory_space=pltpu.VMEM),
            pl.BlockSpec(memory_space=pltpu.VMEM),
        ],
        out_specs=pl.BlockSpec(memory_space=pltpu.VMEM),
        scratch_shapes=[
            pltpu.VMEM((m, k_loc), jnp.bfloat16),
            pltpu.VMEM((k_loc, n), jnp.bfloat16),
            pltpu.VMEM((m, n), jnp.bfloat16),
        ] + [
            buf
            for cs in c_sizes
            for buf in (
                pltpu.VMEM((4, mc, cs), jnp.bfloat16),
                pltpu.VMEM((2, mc, cs), jnp.bfloat16),
                pltpu.VMEM((mc, cs), jnp.bfloat16),
            )
        ] + [
            pltpu.SemaphoreType.DMA((len(STRIPES), 4)),
            pltpu.SemaphoreType.DMA((len(STRIPES), 4)),
        ],
        compiler_params=pltpu.CompilerParams(
            collective_id=0, vmem_limit_bytes=100 * 1024 * 1024,
        ),
    )(x, w_mat)


# device time: 75235 ns/iter; 2.5594x vs baseline; 1.1167x over previous
import jax
import jax.numpy as jnp
from jax import lax
from jax.experimental import pallas as pl
from jax.experimental.pallas import tpu as pltpu

N_DEV = 8

STRIPES = (
    (0, 768, (4, 3, 1)),
    (768, 640, (3, 1, 4)),
    (1408, 640, (1, 4, 3)),
)


def kernel(x, w_mat):
    m, k_loc = x.shape
    _, n = w_mat.shape
    mc = m // N_DEV
    assert n == STRIPES[-1][0] + STRIPES[-1][1]

    def body(x_ref, w_ref, out_ref, w_bf_ref, p_ref,
             r1_0, r2_0, r3_0, r1_1, r2_1, r3_1, r1_2, r2_2, r3_2,
             send_sems, recv_sems):
        r1 = (r1_0, r1_1, r1_2)
        r2 = (r2_0, r2_1, r2_2)
        r3 = (r3_0, r3_1, r3_2)

        my = lax.axis_index("i")

        def partner(g):
            return jnp.bitwise_xor(my, g)

        barrier_sem = pltpu.get_barrier_semaphore()
        for g in (1, 3, 4):
            pl.semaphore_signal(
                barrier_sem, inc=1,
                device_id=(partner(g),), device_id_type=pl.DeviceIdType.MESH,
            )
        pl.semaphore_wait(barrier_sem, 3)

        w_bf_ref[:, :] = w_ref[:, :].astype(jnp.bfloat16)

        def d_list(s):
            _, _, (g1, g2, g3) = STRIPES[s]
            return (0, g3, g2, g2 ^ g3)

        def dot_chunk(s, c):
            c0, cs, _ = STRIPES[s]
            p_ref[pl.ds(c * mc, mc), pl.ds(c0, cs)] = jnp.dot(
                x_ref[pl.ds(c * mc, mc), :].astype(jnp.bfloat16),
                w_bf_ref[:, c0:c0 + cs],
                preferred_element_type=jnp.float32,
            ).astype(jnp.bfloat16)

        def copy(src, dst, s, sem_idx, g):
            return pltpu.make_async_remote_copy(
                src_ref=src, dst_ref=dst,
                send_sem=send_sems.at[s, sem_idx],
                recv_sem=recv_sems.at[s, sem_idx],
                device_id=(partner(g),), device_id_type=pl.DeviceIdType.MESH,
            )

        def fold1(s, j):
            c0, cs, _ = STRIPES[s]
            c = jnp.bitwise_xor(my, d_list(s)[j])
            mine = p_ref[pl.ds(c * mc, mc), pl.ds(c0, cs)]
            r1[s][j] = (mine.astype(jnp.float32)
                        + r1[s][j].astype(jnp.float32)).astype(jnp.bfloat16)

        def fold2(s, j):
            r2[s][j] = (r1[s][j].astype(jnp.float32)
                        + r2[s][j].astype(jnp.float32)).astype(jnp.bfloat16)

        l1 = {}
        for s, (c0, cs, (g1, g2, g3)) in enumerate(STRIPES):
            for j in (2, 3, 1, 0):
                c = jnp.bitwise_xor(partner(g1), d_list(s)[j])
                dot_chunk(s, c)
                rdma = copy(
                    p_ref.at[pl.ds(c * mc, mc), pl.ds(c0, cs)],
                    r1[s].at[j], s, j, g1,
                )
                rdma.start()
                l1[s, j] = rdma

        for j in (2, 3):
            for s in range(3):
                dot_chunk(s, jnp.bitwise_xor(my, d_list(s)[j]))
        for s in range(3):
            dot_chunk(s, jnp.bitwise_xor(my, d_list(s)[1]))
        for s in range(3):
            dot_chunk(s, my)

        l2 = {}
        for s, (c0, cs, (g1, g2, g3)) in enumerate(STRIPES):
            l1[s, 2].wait()
            l1[s, 3].wait()
            fold1(s, 2)
            fold1(s, 3)
            l2[s, 1] = copy(r1[s].at[3], r2[s].at[1], s, 5, g2)
            l2[s, 1].start()
            l2[s, 0] = copy(r1[s].at[2], r2[s].at[0], s, 4, g2)
            l2[s, 0].start()

        l3 = {}
        for s, (c0, cs, (g1, g2, g3)) in enumerate(STRIPES):
            l1[s, 1].wait()
            fold1(s, 1)
            l2[s, 1].wait()
            fold2(s, 1)
            l3[s] = copy(r2[s].at[1], r3[s], s, 6, g3)
            l3[s].start()

        for s, (c0, cs, _) in enumerate(STRIPES):
            l1[s, 0].wait()
            fold1(s, 0)
            l2[s, 0].wait()
            fold2(s, 0)
        for s, (c0, cs, _) in enumerate(STRIPES):
            l3[s].wait()
            out_ref[:, c0:c0 + cs] = (r2[s][0].astype(jnp.float32)
                                      + r3[s][:, :].astype(jnp.float32))

    c_sizes = [cs for _, cs, _ in STRIPES]
    return pl.pallas_call(
        body,
        out_shape=jax.ShapeDtypeStruct((mc, n), jnp.float32),
        in_specs=[
            pl.BlockSpec(memory_space=pltpu.VMEM),
            pl.BlockSpec(memory_space=pltpu.VMEM),
        ],
        out_specs=pl.BlockSpec(memory_space=pltpu.VMEM),
        scratch_shapes=[
            pltpu.VMEM((k_loc, n), jnp.bfloat16),
            pltpu.VMEM((m, n), jnp.bfloat16),
        ] + [
            buf
            for cs in c_sizes
            for buf in (
                pltpu.VMEM((4, mc, cs), jnp.bfloat16),
                pltpu.VMEM((2, mc, cs), jnp.bfloat16),
                pltpu.VMEM((mc, cs), jnp.bfloat16),
            )
        ] + [
            pltpu.SemaphoreType.DMA((len(STRIPES), 7)),
            pltpu.SemaphoreType.DMA((len(STRIPES), 7)),
        ],
        compiler_params=pltpu.CompilerParams(
            collective_id=0, vmem_limit_bytes=100 * 1024 * 1024,
        ),
    )(x, w_mat)


# device time: 73808 ns/iter; 2.6089x vs baseline; 1.0193x over previous
import jax
import jax.numpy as jnp
from jax import lax
from jax.experimental import pallas as pl
from jax.experimental.pallas import tpu as pltpu

N_DEV = 8

STRIPES = (
    (0, 768, (4, 3, 1)),
    (768, 640, (3, 1, 4)),
    (1408, 640, (1, 4, 3)),
)


def kernel(x, w_mat):
    m, k_loc = x.shape
    _, n = w_mat.shape
    mc = m // N_DEV
    assert n == STRIPES[-1][0] + STRIPES[-1][1]

    use_order = []
    for _, _, (g1, g2, g3) in STRIPES:
        for d in (g2, g2 ^ g3, g3, 0):
            use_order.append(g1 ^ d)
    for j in (2, 3):
        for _, _, (g1, g2, g3) in STRIPES:
            use_order.append((0, g3, g2, g2 ^ g3)[j])
    for _, _, (g1, g2, g3) in STRIPES:
        use_order.append(g3)
    use_order.append(0)
    x_order = tuple(dict.fromkeys(use_order))
    assert len(x_order) == N_DEV

    def body(x_ref, w_ref, out_ref, x_vmem, w_vmem, w_bf_ref, p_ref, o_vmem,
             r1_0, r2_0, r3_0, r1_1, r2_1, r3_1, r1_2, r2_2, r3_2,
             x_sems, w_sems, o_sems, send_sems, recv_sems):
        r1 = (r1_0, r1_1, r1_2)
        r2 = (r2_0, r2_1, r2_2)
        r3 = (r3_0, r3_1, r3_2)

        my = lax.axis_index("i")

        def partner(g):
            return jnp.bitwise_xor(my, g)

        barrier_sem = pltpu.get_barrier_semaphore()
        for g in (1, 3, 4):
            pl.semaphore_signal(
                barrier_sem, inc=1,
                device_id=(partner(g),), device_id_type=pl.DeviceIdType.MESH,
            )
        pl.semaphore_wait(barrier_sem, 3)

        x_cp, w_cp = {}, {}
        for s, (c0, cs, _) in enumerate(STRIPES):
            w_cp[s] = pltpu.make_async_copy(
                w_ref.at[:, pl.ds(c0, cs)], w_vmem.at[:, pl.ds(c0, cs)],
                w_sems.at[s],
            )
            w_cp[s].start()
        for idx, e in enumerate(x_order):
            c = jnp.bitwise_xor(my, e)
            x_cp[e] = pltpu.make_async_copy(
                x_ref.at[pl.ds(c * mc, mc), :],
                x_vmem.at[pl.ds(c * mc, mc), :],
                x_sems.at[idx],
            )
            x_cp[e].start()

        x_ready, w_ready = set(), set()

        def ensure_x(e):
            if e not in x_ready:
                x_cp[e].wait()
                x_ready.add(e)

        def ensure_w(s):
            if s not in w_ready:
                w_cp[s].wait()
                c0, cs, _ = STRIPES[s]
                w_bf_ref[:, c0:c0 + cs] = (
                    w_vmem[:, c0:c0 + cs].astype(jnp.bfloat16))
                w_ready.add(s)

        def d_list(s):
            _, _, (g1, g2, g3) = STRIPES[s]
            return (0, g3, g2, g2 ^ g3)

        def dot_chunk(s, e):
            ensure_w(s)
            ensure_x(e)
            c = jnp.bitwise_xor(my, e)
            c0, cs, _ = STRIPES[s]
            p_ref[pl.ds(c * mc, mc), pl.ds(c0, cs)] = jnp.dot(
                x_vmem[pl.ds(c * mc, mc), :].astype(jnp.bfloat16),
                w_bf_ref[:, c0:c0 + cs],
                preferred_element_type=jnp.float32,
            ).astype(jnp.bfloat16)

        def copy(src, dst, s, sem_idx, g):
            return pltpu.make_async_remote_copy(
                src_ref=src, dst_ref=dst,
                send_sem=send_sems.at[s, sem_idx],
                recv_sem=recv_sems.at[s, sem_idx],
                device_id=(partner(g),), device_id_type=pl.DeviceIdType.MESH,
            )

        def fold1(s, j):
            c0, cs, _ = STRIPES[s]
            c = jnp.bitwise_xor(my, d_list(s)[j])
            mine = p_ref[pl.ds(c * mc, mc), pl.ds(c0, cs)]
            r1[s][j] = (mine.astype(jnp.float32)
                        + r1[s][j].astype(jnp.float32)).astype(jnp.bfloat16)

        def fold2(s, j):
            r2[s][j] = (r1[s][j].astype(jnp.float32)
                        + r2[s][j].astype(jnp.float32)).astype(jnp.bfloat16)

        l1 = {}
        for s, (c0, cs, (g1, g2, g3)) in enumerate(STRIPES):
            for j in (2, 3, 1, 0):
                e = g1 ^ d_list(s)[j]
                dot_chunk(s, e)
                c = jnp.bitwise_xor(my, e)
                rdma = copy(
                    p_ref.at[pl.ds(c * mc, mc), pl.ds(c0, cs)],
                    r1[s].at[j], s, j, g1,
                )
                rdma.start()
                l1[s, j] = rdma

        for j in (2, 3):
            for s in range(3):
                dot_chunk(s, d_list(s)[j])
        for s in range(3):
            dot_chunk(s, d_list(s)[1])
        for s in range(3):
            dot_chunk(s, 0)

        l2 = {}
        for s, (c0, cs, (g1, g2, g3)) in enumerate(STRIPES):
            l1[s, 2].wait()
            l1[s, 3].wait()
            fold1(s, 2)
            fold1(s, 3)
            l2[s, 1] = copy(r1[s].at[3], r2[s].at[1], s, 5, g2)
            l2[s, 1].start()
            l2[s, 0] = copy(r1[s].at[2], r2[s].at[0], s, 4, g2)
            l2[s, 0].start()

        l3 = {}
        for s, (c0, cs, (g1, g2, g3)) in enumerate(STRIPES):
            l1[s, 1].wait()
            fold1(s, 1)
            l2[s, 1].wait()
            fold2(s, 1)
            l3[s] = copy(r2[s].at[1], r3[s], s, 6, g3)
            l3[s].start()

        o_cp = {}
        for s, (c0, cs, _) in enumerate(STRIPES):
            l1[s, 0].wait()
            fold1(s, 0)
            l2[s, 0].wait()
            fold2(s, 0)
        for s, (c0, cs, _) in enumerate(STRIPES):
            l3[s].wait()
            o_vmem[:, c0:c0 + cs] = (r2[s][0].astype(jnp.float32)
                                     + r3[s][:, :].astype(jnp.float32))
            o_cp[s] = pltpu.make_async_copy(
                o_vmem.at[:, pl.ds(c0, cs)], out_ref.at[:, pl.ds(c0, cs)],
                o_sems.at[s],
            )
            o_cp[s].start()
        for s in range(3):
            o_cp[s].wait()

    c_sizes = [cs for _, cs, _ in STRIPES]
    return pl.pallas_call(
        body,
        out_shape=jax.ShapeDtypeStruct((mc, n), jnp.float32),
        in_specs=[
            pl.BlockSpec(memory_space=pltpu.MemorySpace.HBM),
            pl.BlockSpec(memory_space=pltpu.MemorySpace.HBM),
        ],
        out_specs=pl.BlockSpec(memory_space=pltpu.MemorySpace.HBM),
        scratch_shapes=[
            pltpu.VMEM((m, k_loc), jnp.float32),
            pltpu.VMEM((k_loc, n), jnp.float32),
            pltpu.VMEM((k_loc, n), jnp.bfloat16),
            pltpu.VMEM((m, n), jnp.bfloat16),
            pltpu.VMEM((mc, n), jnp.float32),
        ] + [
            buf
            for cs in c_sizes
            for buf in (
                pltpu.VMEM((4, mc, cs), jnp.bfloat16),
                pltpu.VMEM((2, mc, cs), jnp.bfloat16),
                pltpu.VMEM((mc, cs), jnp.bfloat16),
            )
        ] + [
            pltpu.SemaphoreType.DMA((N_DEV,)),
            pltpu.SemaphoreType.DMA((len(STRIPES),)),
            pltpu.SemaphoreType.DMA((len(STRIPES),)),
            pltpu.SemaphoreType.DMA((len(STRIPES), 7)),
            pltpu.SemaphoreType.DMA((len(STRIPES), 7)),
        ],
        compiler_params=pltpu.CompilerParams(
            collective_id=0, vmem_limit_bytes=110 * 1024 * 1024,
        ),
    )(x, w_mat)


# device time: 70775 ns/iter; 2.7207x vs baseline; 1.0429x over previous
import jax
import jax.numpy as jnp
from jax import lax
from jax.experimental import pallas as pl
from jax.experimental.pallas import tpu as pltpu

N_DEV = 8

STRIPES = (
    (0, 768, (4, 3, 1)),
    (768, 640, (3, 1, 4)),
    (1408, 640, (1, 4, 3)),
)


def kernel(x, w_mat):
    m, k_loc = x.shape
    _, n = w_mat.shape
    mc = m // N_DEV
    assert n == STRIPES[-1][0] + STRIPES[-1][1]

    use_order = []
    for d_idx in (3, 2, 1, 0):
        for _, _, (g1, g2, g3) in STRIPES:
            use_order.append(g1 ^ (0, g3, g2, g2 ^ g3)[d_idx])
    for j in (3, 2):
        for _, _, (g1, g2, g3) in STRIPES:
            use_order.append((0, g3, g2, g2 ^ g3)[j])
    for _, _, (g1, g2, g3) in STRIPES:
        use_order.append(g3)
    use_order.append(0)
    x_order = tuple(dict.fromkeys(use_order))
    assert len(x_order) == N_DEV

    def body(x_ref, w_ref, out_ref, x_vmem, w_vmem, w_bf_ref, p_ref, o_vmem,
             r1_0, r2_0, r3_0, r1_1, r2_1, r3_1, r1_2, r2_2, r3_2,
             x_sems, w_sems, o_sems, send_sems, recv_sems):
        r1 = (r1_0, r1_1, r1_2)
        r2 = (r2_0, r2_1, r2_2)
        r3 = (r3_0, r3_1, r3_2)

        my = lax.axis_index("i")

        def partner(g):
            return jnp.bitwise_xor(my, g)

        barrier_sem = pltpu.get_barrier_semaphore()
        for g in (1, 3, 4):
            pl.semaphore_signal(
                barrier_sem, inc=1,
                device_id=(partner(g),), device_id_type=pl.DeviceIdType.MESH,
            )
        pl.semaphore_wait(barrier_sem, 3)

        x_cp, w_cp = {}, {}
        for s, (c0, cs, _) in enumerate(STRIPES):
            w_cp[s] = pltpu.make_async_copy(
                w_ref.at[:, pl.ds(c0, cs)], w_vmem.at[:, pl.ds(c0, cs)],
                w_sems.at[s],
            )
            w_cp[s].start()
        for idx, e in enumerate(x_order):
            c = jnp.bitwise_xor(my, e)
            x_cp[e] = pltpu.make_async_copy(
                x_ref.at[pl.ds(c * mc, mc), :],
                x_vmem.at[pl.ds(c * mc, mc), :],
                x_sems.at[idx],
            )
            x_cp[e].start()

        x_ready, w_ready = set(), set()

        def ensure_x(e):
            if e not in x_ready:
                x_cp[e].wait()
                x_ready.add(e)

        def ensure_w(s):
            if s not in w_ready:
                w_cp[s].wait()
                c0, cs, _ = STRIPES[s]
                w_bf_ref[:, c0:c0 + cs] = (
                    w_vmem[:, c0:c0 + cs].astype(jnp.bfloat16))
                w_ready.add(s)

        def d_list(s):
            _, _, (g1, g2, g3) = STRIPES[s]
            return (0, g3, g2, g2 ^ g3)

        def dot_chunk(s, e):
            ensure_w(s)
            ensure_x(e)
            c = jnp.bitwise_xor(my, e)
            c0, cs, _ = STRIPES[s]
            p_ref[pl.ds(c * mc, mc), pl.ds(c0, cs)] = jnp.dot(
                x_vmem[pl.ds(c * mc, mc), :].astype(jnp.bfloat16),
                w_bf_ref[:, c0:c0 + cs],
                preferred_element_type=jnp.float32,
            ).astype(jnp.bfloat16)

        def copy(src, dst, s, sem_idx, g):
            return pltpu.make_async_remote_copy(
                src_ref=src, dst_ref=dst,
                send_sem=send_sems.at[s, sem_idx],
                recv_sem=recv_sems.at[s, sem_idx],
                device_id=(partner(g),), device_id_type=pl.DeviceIdType.MESH,
            )

        def fold1(s, j):
            c0, cs, _ = STRIPES[s]
            c = jnp.bitwise_xor(my, d_list(s)[j])
            mine = p_ref[pl.ds(c * mc, mc), pl.ds(c0, cs)]
            r1[s][j] = (mine.astype(jnp.float32)
                        + r1[s][j].astype(jnp.float32)).astype(jnp.bfloat16)

        def fold2(s, j):
            r2[s][j] = (r1[s][j].astype(jnp.float32)
                        + r2[s][j].astype(jnp.float32)).astype(jnp.bfloat16)

        l1 = {}
        for j in (3, 2, 1, 0):
            for s, (c0, cs, (g1, g2, g3)) in enumerate(STRIPES):
                e = g1 ^ d_list(s)[j]
                dot_chunk(s, e)
                c = jnp.bitwise_xor(my, e)
                rdma = copy(
                    p_ref.at[pl.ds(c * mc, mc), pl.ds(c0, cs)],
                    r1[s].at[j], s, j, g1,
                )
                rdma.start()
                l1[s, j] = rdma

        for j in (3, 2):
            for s in range(3):
                dot_chunk(s, d_list(s)[j])
        for s in range(3):
            dot_chunk(s, d_list(s)[1])
        for s in range(3):
            dot_chunk(s, 0)

        l2 = {}
        for s, (c0, cs, (g1, g2, g3)) in enumerate(STRIPES):
            l1[s, 3].wait()
            fold1(s, 3)
            l2[s, 1] = copy(r1[s].at[3], r2[s].at[1], s, 5, g2)
            l2[s, 1].start()
        for s, (c0, cs, (g1, g2, g3)) in enumerate(STRIPES):
            l1[s, 2].wait()
            fold1(s, 2)
            l2[s, 0] = copy(r1[s].at[2], r2[s].at[0], s, 4, g2)
            l2[s, 0].start()

        l3 = {}
        for s, (c0, cs, (g1, g2, g3)) in enumerate(STRIPES):
            l1[s, 1].wait()
            fold1(s, 1)
            l2[s, 1].wait()
            fold2(s, 1)
            l3[s] = copy(r2[s].at[1], r3[s], s, 6, g3)
            l3[s].start()

        o_cp = {}
        for s, (c0, cs, _) in enumerate(STRIPES):
            l1[s, 0].wait()
            fold1(s, 0)
            l2[s, 0].wait()
            fold2(s, 0)
        for s, (c0, cs, _) in enumerate(STRIPES):
            l3[s].wait()
            o_vmem[:, c0:c0 + cs] = (r2[s][0].astype(jnp.float32)
                                     + r3[s][:, :].astype(jnp.float32))
            o_cp[s] = pltpu.make_async_copy(
                o_vmem.at[:, pl.ds(c0, cs)], out_ref.at[:, pl.ds(c0, cs)],
                o_sems.at[s],
            )
            o_cp[s].start()
        for s in range(3):
            o_cp[s].wait()

    c_sizes = [cs for _, cs, _ in STRIPES]
    return pl.pallas_call(
        body,
        out_shape=jax.ShapeDtypeStruct((mc, n), jnp.float32),
        in_specs=[
            pl.BlockSpec(memory_space=pltpu.MemorySpace.HBM),
            pl.BlockSpec(memory_space=pltpu.MemorySpace.HBM),
        ],
        out_specs=pl.BlockSpec(memory_space=pltpu.MemorySpace.HBM),
        scratch_shapes=[
            pltpu.VMEM((m, k_loc), jnp.float32),
            pltpu.VMEM((k_loc, n), jnp.float32),
            pltpu.VMEM((k_loc, n), jnp.bfloat16),
            pltpu.VMEM((m, n), jnp.bfloat16),
            pltpu.VMEM((mc, n), jnp.float32),
        ] + [
            buf
            for cs in c_sizes
            for buf in (
                pltpu.VMEM((4, mc, cs), jnp.bfloat16),
                pltpu.VMEM((2, mc, cs), jnp.bfloat16),
                pltpu.VMEM((mc, cs), jnp.bfloat16),
            )
        ] + [
            pltpu.SemaphoreType.DMA((N_DEV,)),
            pltpu.SemaphoreType.DMA((len(STRIPES),)),
            pltpu.SemaphoreType.DMA((len(STRIPES),)),
            pltpu.SemaphoreType.DMA((len(STRIPES), 7)),
            pltpu.SemaphoreType.DMA((len(STRIPES), 7)),
        ],
        compiler_params=pltpu.CompilerParams(
            collective_id=0, vmem_limit_bytes=110 * 1024 * 1024,
        ),
    )(x, w_mat)


# device time: 68533 ns/iter; 2.8097x vs baseline; 1.0327x over previous
import jax
import jax.numpy as jnp
from jax import lax
from jax.experimental import pallas as pl
from jax.experimental.pallas import tpu as pltpu

N_DEV = 8

STRIPES = (
    (0, 768, (4, 3, 1)),
    (768, 640, (3, 1, 4)),
    (1408, 640, (1, 4, 3)),
)


def kernel(x, w_mat):
    m, k_loc = x.shape
    _, n = w_mat.shape
    mc = m // N_DEV
    assert n == STRIPES[-1][0] + STRIPES[-1][1]

    use_order = []
    for d_idx in (3, 2, 1, 0):
        for _, _, (g1, g2, g3) in STRIPES:
            use_order.append(g1 ^ (0, g3, g2, g2 ^ g3)[d_idx])
    for j in (3, 2):
        for _, _, (g1, g2, g3) in STRIPES:
            use_order.append((0, g3, g2, g2 ^ g3)[j])
    for _, _, (g1, g2, g3) in STRIPES:
        use_order.append(g3)
    use_order.append(0)
    x_order = tuple(dict.fromkeys(use_order))
    assert len(x_order) == N_DEV

    def body(x_ref, w_ref, out_ref, x_vmem, w_vmem, w_bf_ref, p_ref, o_vmem,
             r1_0, r2_0, r3_0, r1_1, r2_1, r3_1, r1_2, r2_2, r3_2,
             x_sems, w_sems, o_sems, send_sems, recv_sems):
        r1 = (r1_0, r1_1, r1_2)
        r2 = (r2_0, r2_1, r2_2)
        r3 = (r3_0, r3_1, r3_2)

        my = lax.axis_index("i")

        def partner(g):
            return jnp.bitwise_xor(my, g)

        x_cp, w_cp = {}, {}
        for s, (c0, cs, _) in enumerate(STRIPES):
            w_cp[s] = pltpu.make_async_copy(
                w_ref.at[:, pl.ds(c0, cs)], w_vmem.at[:, pl.ds(c0, cs)],
                w_sems.at[s],
            )
            w_cp[s].start()
        for idx, e in enumerate(x_order):
            c = jnp.bitwise_xor(my, e)
            x_cp[e] = pltpu.make_async_copy(
                x_ref.at[pl.ds(c * mc, mc), :],
                x_vmem.at[pl.ds(c * mc, mc), :],
                x_sems.at[idx],
            )
            x_cp[e].start()

        barrier_sem = pltpu.get_barrier_semaphore()
        for g in (1, 3, 4):
            pl.semaphore_signal(
                barrier_sem, inc=1,
                device_id=(partner(g),), device_id_type=pl.DeviceIdType.MESH,
            )
        pl.semaphore_wait(barrier_sem, 3)

        x_ready, w_ready = set(), set()

        def ensure_x(e):
            if e not in x_ready:
                x_cp[e].wait()
                x_ready.add(e)

        def ensure_w(s):
            if s not in w_ready:
                w_cp[s].wait()
                c0, cs, _ = STRIPES[s]
                w_bf_ref[:, c0:c0 + cs] = (
                    w_vmem[:, c0:c0 + cs].astype(jnp.bfloat16))
                w_ready.add(s)

        def d_list(s):
            _, _, (g1, g2, g3) = STRIPES[s]
            return (0, g3, g2, g2 ^ g3)

        def dot_chunk(s, e):
            ensure_w(s)
            ensure_x(e)
            c = jnp.bitwise_xor(my, e)
            c0, cs, _ = STRIPES[s]
            p_ref[pl.ds(c * mc, mc), pl.ds(c0, cs)] = jnp.dot(
                x_vmem[pl.ds(c * mc, mc), :].astype(jnp.bfloat16),
                w_bf_ref[:, c0:c0 + cs],
                preferred_element_type=jnp.float32,
            ).astype(jnp.bfloat16)

        def copy(src, dst, s, sem_idx, g):
            return pltpu.make_async_remote_copy(
                src_ref=src, dst_ref=dst,
                send_sem=send_sems.at[s, sem_idx],
                recv_sem=recv_sems.at[s, sem_idx],
                device_id=(partner(g),), device_id_type=pl.DeviceIdType.MESH,
            )

        def fold1(s, j):
            c0, cs, _ = STRIPES[s]
            c = jnp.bitwise_xor(my, d_list(s)[j])
            mine = p_ref[pl.ds(c * mc, mc), pl.ds(c0, cs)]
            r1[s][j] = (mine.astype(jnp.float32)
                        + r1[s][j].astype(jnp.float32)).astype(jnp.bfloat16)

        def fold2(s, j):
            r2[s][j] = (r1[s][j].astype(jnp.float32)
                        + r2[s][j].astype(jnp.float32)).astype(jnp.bfloat16)

        l1 = {}
        for j in (3, 2, 1, 0):
            for s, (c0, cs, (g1, g2, g3)) in enumerate(STRIPES):
                e = g1 ^ d_list(s)[j]
                dot_chunk(s, e)
                c = jnp.bitwise_xor(my, e)
                rdma = copy(
                    p_ref.at[pl.ds(c * mc, mc), pl.ds(c0, cs)],
                    r1[s].at[j], s, j, g1,
                )
                rdma.start()
                l1[s, j] = rdma

        for j in (3, 2):
            for s in range(3):
                dot_chunk(s, d_list(s)[j])
        for s in range(3):
            dot_chunk(s, d_list(s)[1])
        for s in range(3):
            dot_chunk(s, 0)

        l2 = {}
        for s, (c0, cs, (g1, g2, g3)) in enumerate(STRIPES):
            l1[s, 3].wait()
            fold1(s, 3)
            l2[s, 1] = copy(r1[s].at[3], r2[s].at[1], s, 5, g2)
            l2[s, 1].start()
        for s, (c0, cs, (g1, g2, g3)) in enumerate(STRIPES):
            l1[s, 2].wait()
            fold1(s, 2)
            l2[s, 0] = copy(r1[s].at[2], r2[s].at[0], s, 4, g2)
            l2[s, 0].start()

        l3 = {}
        for s, (c0, cs, (g1, g2, g3)) in enumerate(STRIPES):
            l1[s, 1].wait()
            fold1(s, 1)
            l2[s, 1].wait()
            fold2(s, 1)
            l3[s] = copy(r2[s].at[1], r3[s], s, 6, g3)
            l3[s].start()

        o_cp = {}
        for s, (c0, cs, _) in enumerate(STRIPES):
            l1[s, 0].wait()
            fold1(s, 0)
            l2[s, 0].wait()
            fold2(s, 0)
        for s, (c0, cs, _) in enumerate(STRIPES):
            l3[s].wait()
            o_vmem[:, c0:c0 + cs] = (r2[s][0].astype(jnp.float32)
                                     + r3[s][:, :].astype(jnp.float32))
            o_cp[s] = pltpu.make_async_copy(
                o_vmem.at[:, pl.ds(c0, cs)], out_ref.at[:, pl.ds(c0, cs)],
                o_sems.at[s],
            )
            o_cp[s].start()
        for s in range(3):
            o_cp[s].wait()

    c_sizes = [cs for _, cs, _ in STRIPES]
    return pl.pallas_call(
        body,
        out_shape=jax.ShapeDtypeStruct((mc, n), jnp.float32),
        in_specs=[
            pl.BlockSpec(memory_space=pltpu.MemorySpace.HBM),
            pl.BlockSpec(memory_space=pltpu.MemorySpace.HBM),
        ],
        out_specs=pl.BlockSpec(memory_space=pltpu.MemorySpace.HBM),
        scratch_shapes=[
            pltpu.VMEM((m, k_loc), jnp.float32),
            pltpu.VMEM((k_loc, n), jnp.float32),
            pltpu.VMEM((k_loc, n), jnp.bfloat16),
            pltpu.VMEM((m, n), jnp.bfloat16),
            pltpu.VMEM((mc, n), jnp.float32),
        ] + [
            buf
            for cs in c_sizes
            for buf in (
                pltpu.VMEM((4, mc, cs), jnp.bfloat16),
                pltpu.VMEM((2, mc, cs), jnp.bfloat16),
                pltpu.VMEM((mc, cs), jnp.bfloat16),
            )
        ] + [
            pltpu.SemaphoreType.DMA((N_DEV,)),
            pltpu.SemaphoreType.DMA((len(STRIPES),)),
            pltpu.SemaphoreType.DMA((len(STRIPES),)),
            pltpu.SemaphoreType.DMA((len(STRIPES), 7)),
            pltpu.SemaphoreType.DMA((len(STRIPES), 7)),
        ],
        compiler_params=pltpu.CompilerParams(
            collective_id=0, vmem_limit_bytes=110 * 1024 * 1024,
        ),
    )(x, w_mat)
